# baseline (speedup 1.0000x reference)
import functools

import jax
import jax.numpy as jnp
from jax import lax
from jax.experimental import pallas as pl
from jax.experimental.pallas import tpu as pltpu
from jax.experimental.pallas import tpu_sc as plsc

N = 10000
E = 320000
D = 128
DE = 16

NC = 2
NS = 16
CF = 64

N_PAD = 10240
ROWS_PER_TILE = N_PAD // NS

BE = 2560

NLOC = (E // CF) // NC // NS
NGRP = NLOC // 2
CHUNKS_PER_CORE = (E // CF) // NC


def _xw_body(x_ref, w_ref, b_ref, o_ref):
    o_ref[...] = (
        jnp.dot(x_ref[...], w_ref[...], preferred_element_type=jnp.float32)
        + b_ref[...]
    )


def _eaw_body(ea_ref, w_ref, o_ref):
    o_ref[...] = jnp.dot(ea_ref[...], w_ref[...], preferred_element_type=jnp.float32)


def _node_body(p_ref, wn_ref, bn_ref, gn_ref, btn_ref, o_ref):
    ft = p_ref[0, :N, :] + p_ref[1, :N, :]
    f = jnp.dot(ft, wn_ref[...], preferred_element_type=jnp.float32) + bn_ref[...]
    m = jnp.mean(f, axis=-1, keepdims=True)
    cgap = f - m
    v = jnp.mean(cgap * cgap, axis=-1, keepdims=True)
    h = cgap * lax.rsqrt(v + 1e-5) * gn_ref[...] + btn_ref[...]
    o_ref[...] = jnp.maximum(h, 0.0)


_sc_mesh = plsc.VectorSubcoreMesh(core_axis_name="c", subcore_axis_name="s")


@functools.partial(
    pl.kernel,
    out_type=jax.ShapeDtypeStruct((NC, N_PAD, D), jnp.float32),
    mesh=_sc_mesh,
    scratch_types=[
        pltpu.VMEM((2, CF), jnp.int32),
        pltpu.VMEM((2, CF), jnp.int32),
        pltpu.VMEM((2, CF, D), jnp.float32),
        pltpu.VMEM((2, CF, D), jnp.float32),
        pltpu.VMEM((D,), jnp.float32),
        pltpu.VMEM((D,), jnp.float32),
        pltpu.VMEM_SHARED((N_PAD, D), jnp.float32),
        pltpu.SemaphoreType.DMA((2,)),
        pltpu.SemaphoreType.DMA((2,)),
        pltpu.SemaphoreType.DMA((2,)),
        pltpu.SemaphoreType.DMA((2,)),
        pltpu.SemaphoreType.DMA((2,)),
    ],
)
def _fused_sc(
    xw_hbm, eaw_hbm, row_hbm, col_hbm, g1_hbm, bt1_hbm, zeros_hbm, out_hbm,
    rix, cix, gbuf, ebuf, g1v, btv, flow_sh,
    sem_r, sem_c, sem_g, sem_e, sem_sc,
):
    cid = lax.axis_index("c")
    sid = lax.axis_index("s")

    pltpu.sync_copy(g1_hbm, g1v)
    pltpu.sync_copy(bt1_hbm, btv)
    pltpu.sync_copy(zeros_hbm, flow_sh.at[pl.ds(sid * ROWS_PER_TILE, ROWS_PER_TILE)])
    plsc.subcore_barrier()

    g1r = [g1v[pl.ds(16 * k, 16)] for k in range(D // 16)]
    btr = [btv[pl.ds(16 * k, 16)] for k in range(D // 16)]

    def base_of(j):
        return (cid * CHUNKS_PER_CORE + sid + j * NS) * CF

    def fire_loads(j, b):
        base = base_of(j)
        pltpu.async_copy(row_hbm.at[pl.ds(base, CF)], rix.at[b], sem_r.at[b])
        pltpu.async_copy(col_hbm.at[pl.ds(base, CF)], cix.at[b], sem_c.at[b])
        pltpu.async_copy(eaw_hbm.at[pl.ds(base, CF)], ebuf.at[b], sem_e.at[b])

    def wait_sem(src, dst, sem):
        pltpu.make_async_copy(src, dst, sem).wait()

    def select_trash(b):
        for ii in range(CF // 16):
            r = rix[b, pl.ds(ii * 16, 16)]
            cc = cix[b, pl.ds(ii * 16, 16)]
            trash = jnp.full((16,), N, jnp.int32)
            rix[b, pl.ds(ii * 16, 16)] = jnp.where(r == cc, trash, r)

    def compute_ln(b):
        gb = gbuf.at[b]
        eb = ebuf.at[b]
        half = jnp.full((16,), 0.5, jnp.float32)
        thr = jnp.full((16,), 1.5, jnp.float32)
        magic = jnp.full((16,), 0x5F3759DF, jnp.int32)
        invd = jnp.full((16,), 1.0 / D, jnp.float32)
        eps = jnp.full((16,), 1e-5, jnp.float32)
        lanes = lax.iota(jnp.int32, 16)
        perms = [lanes ^ (1 << p) for p in range(4)]

        def _hsum(v):
            for p in perms:
                v = v + v.at[p].get(mode="promise_in_bounds")
            return v

        def stats8(t, carry):
            for u in range(8):
                e = t * 8 + u
                s = None
                q = None
                for k in range(D // 16):
                    fk = gb[e, pl.ds(16 * k, 16)] + eb[e, pl.ds(16 * k, 16)]
                    gb[e, pl.ds(16 * k, 16)] = fk
                    s = fk if s is None else s + fk
                    q = fk * fk if q is None else q + fk * fk
                mm = _hsum(s) * invd
                sq = _hsum(q) * invd
                vv = sq - mm * mm + eps
                y = lax.bitcast_convert_type(
                    magic - (lax.bitcast_convert_type(vv, jnp.int32) >> 1),
                    jnp.float32,
                )
                hv = half * vv
                for _ in range(2):
                    y = y * (thr - hv * y * y)
                eb[e, pl.ds(0, 16)] = y
                eb[e, pl.ds(16, 16)] = mm
            return carry

        def norm8(t, carry):
            for u in range(8):
                e = t * 8 + u
                y = eb[e, pl.ds(0, 16)]
                mm = eb[e, pl.ds(16, 16)]
                for k in range(D // 16):
                    t0 = (gb[e, pl.ds(16 * k, 16)] - mm) * y
                    out = jnp.maximum(t0 * g1r[k] + btr[k], 0.0)
                    gb[e, pl.ds(16 * k, 16)] = out
            return carry

        lax.fori_loop(0, CF // 8, stats8, 0)
        lax.fori_loop(0, CF // 8, norm8, 0)

    def process(j, b):
        wait_sem(col_hbm.at[pl.ds(0, CF)], cix.at[b], sem_c.at[b])
        pltpu.async_copy(xw_hbm.at[cix.at[b]], gbuf.at[b], sem_g.at[b])
        wait_sem(row_hbm.at[pl.ds(0, CF)], rix.at[b], sem_r.at[b])
        select_trash(b)
        wait_sem(eaw_hbm.at[pl.ds(0, CF)], ebuf.at[b], sem_e.at[b])
        wait_sem(xw_hbm.at[cix.at[b]], gbuf.at[b], sem_g.at[b])
        compute_ln(b)
        pltpu.async_copy(gbuf.at[b], flow_sh.at[rix.at[b]], sem_sc.at[b], add=True)

    def drain_scatter(b):
        pltpu.make_async_copy(gbuf.at[b], flow_sh.at[rix.at[b]], sem_sc.at[b]).wait()

    fire_loads(0, 0)

    def body(grp, carry):
        j0 = 2 * grp
        wait_sem(col_hbm.at[pl.ds(0, CF)], cix.at[0], sem_c.at[0])
        pltpu.async_copy(xw_hbm.at[cix.at[0]], gbuf.at[0], sem_g.at[0])

        @pl.when(grp > 0)
        def _():
            drain_scatter(1)

        fire_loads(j0 + 1, 1)
        wait_sem(row_hbm.at[pl.ds(0, CF)], rix.at[0], sem_r.at[0])
        select_trash(0)
        wait_sem(eaw_hbm.at[pl.ds(0, CF)], ebuf.at[0], sem_e.at[0])
        wait_sem(xw_hbm.at[cix.at[0]], gbuf.at[0], sem_g.at[0])
        compute_ln(0)
        pltpu.async_copy(gbuf.at[0], flow_sh.at[rix.at[0]], sem_sc.at[0], add=True)

        wait_sem(col_hbm.at[pl.ds(0, CF)], cix.at[1], sem_c.at[1])
        pltpu.async_copy(xw_hbm.at[cix.at[1]], gbuf.at[1], sem_g.at[1])

        @pl.when(grp < NGRP - 1)
        def _():
            drain_scatter(0)
            fire_loads(j0 + 2, 0)

        wait_sem(row_hbm.at[pl.ds(0, CF)], rix.at[1], sem_r.at[1])
        select_trash(1)
        wait_sem(eaw_hbm.at[pl.ds(0, CF)], ebuf.at[1], sem_e.at[1])
        wait_sem(xw_hbm.at[cix.at[1]], gbuf.at[1], sem_g.at[1])
        compute_ln(1)
        pltpu.async_copy(gbuf.at[1], flow_sh.at[rix.at[1]], sem_sc.at[1], add=True)
        return carry

    lax.fori_loop(0, NGRP, body, 0)
    drain_scatter(0)
    drain_scatter(1)

    nrem = CHUNKS_PER_CORE - NLOC * NS

    @pl.when(sid < nrem)
    def _():
        base = (cid * CHUNKS_PER_CORE + NLOC * NS + sid) * CF
        pltpu.sync_copy(row_hbm.at[pl.ds(base, CF)], rix.at[0])
        pltpu.sync_copy(col_hbm.at[pl.ds(base, CF)], cix.at[0])
        pltpu.sync_copy(eaw_hbm.at[pl.ds(base, CF)], ebuf.at[0])
        select_trash(0)
        pltpu.async_copy(xw_hbm.at[cix.at[0]], gbuf.at[0], sem_g.at[0]).wait()
        compute_ln(0)
        pltpu.sync_copy(gbuf.at[0], flow_sh.at[rix.at[0]], add=True)

    plsc.subcore_barrier()
    pltpu.sync_copy(
        flow_sh.at[pl.ds(sid * ROWS_PER_TILE, ROWS_PER_TILE)],
        out_hbm.at[cid, pl.ds(sid * ROWS_PER_TILE, ROWS_PER_TILE)],
    )


def kernel(x, edge_index, edge_attr, W1, b1, g1, bt1, Wn, bn, gn, btn):
    row = edge_index[0]
    col = edge_index[1]
    W1a = W1[:D]
    W1b = W1[D:]

    xw = pl.pallas_call(
        _xw_body,
        out_shape=jax.ShapeDtypeStruct((N, D), jnp.float32),
    )(x, W1a, b1.reshape(1, D))

    eaw = pl.pallas_call(
        _eaw_body,
        grid=(E // BE,),
        in_specs=[
            pl.BlockSpec((BE, DE), lambda i: (i, 0)),
            pl.BlockSpec((DE, D), lambda i: (0, 0)),
        ],
        out_specs=pl.BlockSpec((BE, D), lambda i: (i, 0)),
        out_shape=jax.ShapeDtypeStruct((E, D), jnp.float32),
    )(edge_attr, W1b)

    zeros = jnp.zeros((ROWS_PER_TILE, D), jnp.float32)
    partials = _fused_sc(xw, eaw, row, col, g1, bt1, zeros)

    out = pl.pallas_call(
        _node_body,
        out_shape=jax.ShapeDtypeStruct((N, D), jnp.float32),
    )(partials, Wn, bn.reshape(1, D), gn.reshape(1, D), btn.reshape(1, D))
    return out


# speedup vs baseline: 1.2932x; 1.2932x over previous
import functools

import jax
import jax.numpy as jnp
from jax import lax
from jax.experimental import pallas as pl
from jax.experimental.pallas import tpu as pltpu
from jax.experimental.pallas import tpu_sc as plsc

N = 10000
E = 320000
D = 128
DE = 16

NC = 2
NS = 16
NW = NC * NS
C = 128

N_PAD = 10112
ROWS_PER_TILE = N_PAD // NS

BE = 2560

K = 6
NGRP = 13
KS = 3
NGRPS = 26


def _xw_body(x_ref, w_ref, b_ref, o_ref):
    o_ref[...] = (
        jnp.dot(x_ref[...], w_ref[...], preferred_element_type=jnp.float32)
        + b_ref[...]
    )


def _edge_body(g_ref, ea_ref, w_ref, g1_ref, bt1_ref, o_ref):
    f = g_ref[...] + jnp.dot(
        ea_ref[...], w_ref[...], preferred_element_type=jnp.float32
    )
    m = jnp.mean(f, axis=-1, keepdims=True)
    cgap = f - m
    v = jnp.mean(cgap * cgap, axis=-1, keepdims=True)
    h = cgap * lax.rsqrt(v + 1e-5) * g1_ref[...] + bt1_ref[...]
    o_ref[...] = jnp.maximum(h, 0.0)


def _node_body(p_ref, wn_ref, bn_ref, gn_ref, btn_ref, o_ref):
    ft = p_ref[0, :N, :] + p_ref[1, :N, :]
    f = jnp.dot(ft, wn_ref[...], preferred_element_type=jnp.float32) + bn_ref[...]
    m = jnp.mean(f, axis=-1, keepdims=True)
    cgap = f - m
    v = jnp.mean(cgap * cgap, axis=-1, keepdims=True)
    h = cgap * lax.rsqrt(v + 1e-5) * gn_ref[...] + btn_ref[...]
    o_ref[...] = jnp.maximum(h, 0.0)


_sc_mesh = plsc.VectorSubcoreMesh(core_axis_name="c", subcore_axis_name="s")


@functools.partial(
    pl.kernel,
    out_type=jax.ShapeDtypeStruct((E, D), jnp.float32),
    mesh=_sc_mesh,
    scratch_types=[
        pltpu.VMEM((K, C), jnp.int32),
        pltpu.VMEM((K, C, D), jnp.float32),
        pltpu.SemaphoreType.DMA((K,)),
        pltpu.SemaphoreType.DMA((K,)),
        pltpu.SemaphoreType.DMA((K,)),
    ],
)
def _gather_sc(xw_hbm, col_hbm, g_hbm, idx_v, rows_v, sem_i, sem_g, sem_s):
    wid = lax.axis_index("s") * NC + lax.axis_index("c")
    nchunks = E // C

    def body(grp, carry):
        for i in range(K):
            chunk = wid + (grp * K + i) * NW
            pltpu.async_copy(
                col_hbm.at[pl.ds(chunk * C, C)], idx_v.at[i], sem_i.at[i]
            )
        for i in range(K):
            @pl.when(grp > 0)
            def _():
                pltpu.make_async_copy(
                    rows_v.at[i], g_hbm.at[pl.ds(0, C)], sem_s.at[i]
                ).wait()
        for i in range(K):
            pltpu.make_async_copy(
                col_hbm.at[pl.ds(0, C)], idx_v.at[i], sem_i.at[i]
            ).wait()
            pltpu.async_copy(xw_hbm.at[idx_v.at[i]], rows_v.at[i], sem_g.at[i])
        for i in range(K):
            chunk = wid + (grp * K + i) * NW
            pltpu.make_async_copy(
                xw_hbm.at[idx_v.at[i]], rows_v.at[i], sem_g.at[i]
            ).wait()
            pltpu.async_copy(
                rows_v.at[i], g_hbm.at[pl.ds(chunk * C, C)], sem_s.at[i]
            )
        return carry

    lax.fori_loop(0, NGRP, body, 0)
    for i in range(K):
        pltpu.make_async_copy(
            rows_v.at[i], g_hbm.at[pl.ds(0, C)], sem_s.at[i]
        ).wait()

    @pl.when(wid < nchunks - NGRP * K * NW)
    def _():
        base = (NGRP * K * NW + wid) * C
        pltpu.sync_copy(col_hbm.at[pl.ds(base, C)], idx_v.at[0])
        pltpu.async_copy(xw_hbm.at[idx_v.at[0]], rows_v.at[0], sem_g.at[0]).wait()
        pltpu.sync_copy(rows_v.at[0], g_hbm.at[pl.ds(base, C)])


@functools.partial(
    pl.kernel,
    out_type=jax.ShapeDtypeStruct((NC, N_PAD, D), jnp.float32),
    mesh=_sc_mesh,
    scratch_types=[
        pltpu.VMEM((2 * KS, C), jnp.int32),
        pltpu.VMEM((KS, C, D), jnp.float32),
        pltpu.VMEM_SHARED((N_PAD, D), jnp.float32),
        pltpu.SemaphoreType.DMA((KS,)),
        pltpu.SemaphoreType.DMA((KS,)),
        pltpu.SemaphoreType.DMA((KS,)),
    ],
)
def _scatter_sc(
    f_hbm, row_hbm, col_hbm, zeros_hbm, out_hbm,
    idx_v, fbuf, flow_sh, sem_i, sem_f, sem_sc,
):
    cid = lax.axis_index("c")
    sid = lax.axis_index("s")

    pltpu.sync_copy(zeros_hbm, flow_sh.at[pl.ds(sid * ROWS_PER_TILE, ROWS_PER_TILE)])
    plsc.subcore_barrier()

    nchunks_half = (E // C) // NC

    def _select_trash(i):
        for ii in range(C // 16):
            r = idx_v[2 * i, pl.ds(ii * 16, 16)]
            cc = idx_v[2 * i + 1, pl.ds(ii * 16, 16)]
            trash = jnp.full((16,), N, jnp.int32)
            idx_v[2 * i, pl.ds(ii * 16, 16)] = jnp.where(r == cc, trash, r)

    def body(grp, carry):
        for i in range(KS):
            @pl.when(grp > 0)
            def _():
                pltpu.make_async_copy(
                    fbuf.at[i], flow_sh.at[idx_v.at[2 * i]], sem_sc.at[i]
                ).wait()
            chunk_local = sid + (grp * KS + i) * NS
            base = (cid * nchunks_half + chunk_local) * C
            pltpu.async_copy(row_hbm.at[pl.ds(base, C)], idx_v.at[2 * i], sem_i.at[i])
            pltpu.async_copy(col_hbm.at[pl.ds(base, C)], idx_v.at[2 * i + 1], sem_i.at[i])
            pltpu.async_copy(f_hbm.at[pl.ds(base, C)], fbuf.at[i], sem_f.at[i])
        for i in range(KS):
            pltpu.make_async_copy(
                row_hbm.at[pl.ds(0, C)], idx_v.at[2 * i], sem_i.at[i]
            ).wait()
            pltpu.make_async_copy(
                col_hbm.at[pl.ds(0, C)], idx_v.at[2 * i + 1], sem_i.at[i]
            ).wait()
            _select_trash(i)
            pltpu.make_async_copy(
                f_hbm.at[pl.ds(0, C)], fbuf.at[i], sem_f.at[i]
            ).wait()
            pltpu.async_copy(
                fbuf.at[i], flow_sh.at[idx_v.at[2 * i]], sem_sc.at[i], add=True
            )
        return carry

    lax.fori_loop(0, NGRPS, body, 0)
    for i in range(KS):
        pltpu.make_async_copy(
            fbuf.at[i], flow_sh.at[idx_v.at[2 * i]], sem_sc.at[i]
        ).wait()

    @pl.when(sid < nchunks_half - NGRPS * KS * NS)
    def _():
        base = (cid * nchunks_half + NGRPS * KS * NS + sid) * C
        pltpu.sync_copy(row_hbm.at[pl.ds(base, C)], idx_v.at[0])
        pltpu.sync_copy(col_hbm.at[pl.ds(base, C)], idx_v.at[1])
        _select_trash(0)
        pltpu.sync_copy(f_hbm.at[pl.ds(base, C)], fbuf.at[0])
        pltpu.sync_copy(fbuf.at[0], flow_sh.at[idx_v.at[0]], add=True)

    plsc.subcore_barrier()
    pltpu.sync_copy(
        flow_sh.at[pl.ds(sid * ROWS_PER_TILE, ROWS_PER_TILE)],
        out_hbm.at[cid, pl.ds(sid * ROWS_PER_TILE, ROWS_PER_TILE)],
    )


def kernel(x, edge_index, edge_attr, W1, b1, g1, bt1, Wn, bn, gn, btn):
    row = edge_index[0]
    col = edge_index[1]
    W1a = W1[:D]
    W1b = W1[D:]

    xw = pl.pallas_call(
        _xw_body,
        out_shape=jax.ShapeDtypeStruct((N, D), jnp.float32),
    )(x, W1a, b1.reshape(1, D))

    g = _gather_sc(xw, col)

    nblk = E // BE
    f = pl.pallas_call(
        _edge_body,
        grid=(nblk,),
        in_specs=[
            pl.BlockSpec((BE, D), lambda i: (i, 0)),
            pl.BlockSpec((BE, DE), lambda i: (i, 0)),
            pl.BlockSpec((DE, D), lambda i: (0, 0)),
            pl.BlockSpec((1, D), lambda i: (0, 0)),
            pl.BlockSpec((1, D), lambda i: (0, 0)),
        ],
        out_specs=pl.BlockSpec((BE, D), lambda i: (i, 0)),
        out_shape=jax.ShapeDtypeStruct((E, D), jnp.float32),
    )(g, edge_attr, W1b, g1.reshape(1, D), bt1.reshape(1, D))

    zeros = jnp.zeros((ROWS_PER_TILE, D), jnp.float32)
    partials = _scatter_sc(f, row, col, zeros)

    out = pl.pallas_call(
        _node_body,
        out_shape=jax.ShapeDtypeStruct((N, D), jnp.float32),
    )(partials, Wn, bn.reshape(1, D), gn.reshape(1, D), btn.reshape(1, D))
    return out


# speedup vs baseline: 1.3634x; 1.0543x over previous
import functools

import jax
import jax.numpy as jnp
from jax import lax
from jax.experimental import pallas as pl
from jax.experimental.pallas import tpu as pltpu
from jax.experimental.pallas import tpu_sc as plsc

N = 10000
E = 320000
D = 128
DE = 16

NC = 2
NS = 16
NW = NC * NS
C = 128

N_PAD = 10112
ROWS_PER_TILE = N_PAD // NS

BE = 4000

K = 6
NGRP = 13
KS = 3
NGRPS = 26


def _xw_body(x_ref, w_ref, b_ref, o_ref):
    o_ref[...] = (
        jnp.dot(x_ref[...], w_ref[...], preferred_element_type=jnp.float32)
        + b_ref[...]
    )


def _edge_body(g_ref, ea_ref, w_ref, g1_ref, bt1_ref, o_ref):
    f = g_ref[...] + jnp.dot(
        ea_ref[...], w_ref[...], preferred_element_type=jnp.float32
    )
    m = jnp.mean(f, axis=-1, keepdims=True)
    cgap = f - m
    v = jnp.mean(cgap * cgap, axis=-1, keepdims=True)
    h = cgap * lax.rsqrt(v + 1e-5) * g1_ref[...] + bt1_ref[...]
    o_ref[...] = jnp.maximum(h, 0.0)


def _node_body(p_ref, wn_ref, bn_ref, gn_ref, btn_ref, o_ref):
    ft = p_ref[0, :N, :] + p_ref[1, :N, :]
    f = jnp.dot(ft, wn_ref[...], preferred_element_type=jnp.float32) + bn_ref[...]
    m = jnp.mean(f, axis=-1, keepdims=True)
    cgap = f - m
    v = jnp.mean(cgap * cgap, axis=-1, keepdims=True)
    h = cgap * lax.rsqrt(v + 1e-5) * gn_ref[...] + btn_ref[...]
    o_ref[...] = jnp.maximum(h, 0.0)


_sc_mesh = plsc.VectorSubcoreMesh(core_axis_name="c", subcore_axis_name="s")


@functools.partial(
    pl.kernel,
    out_type=jax.ShapeDtypeStruct((E, D), jnp.float32),
    mesh=_sc_mesh,
    scratch_types=[
        pltpu.VMEM((K, C), jnp.int32),
        pltpu.VMEM((K, C, D), jnp.float32),
        pltpu.SemaphoreType.DMA((K,)),
        pltpu.SemaphoreType.DMA((K,)),
        pltpu.SemaphoreType.DMA((K,)),
    ],
)
def _gather_sc(xw_hbm, col_hbm, g_hbm, idx_v, rows_v, sem_i, sem_g, sem_s):
    wid = lax.axis_index("s") * NC + lax.axis_index("c")
    nchunks = E // C

    def body(grp, carry):
        for i in range(K):
            chunk = wid + (grp * K + i) * NW
            pltpu.async_copy(
                col_hbm.at[pl.ds(chunk * C, C)], idx_v.at[i], sem_i.at[i]
            )
        for i in range(K):
            @pl.when(grp > 0)
            def _():
                pltpu.make_async_copy(
                    rows_v.at[i], g_hbm.at[pl.ds(0, C)], sem_s.at[i]
                ).wait()
        for i in range(K):
            pltpu.make_async_copy(
                col_hbm.at[pl.ds(0, C)], idx_v.at[i], sem_i.at[i]
            ).wait()
            pltpu.async_copy(xw_hbm.at[idx_v.at[i]], rows_v.at[i], sem_g.at[i])
        for i in range(K):
            chunk = wid + (grp * K + i) * NW
            pltpu.make_async_copy(
                xw_hbm.at[idx_v.at[i]], rows_v.at[i], sem_g.at[i]
            ).wait()
            pltpu.async_copy(
                rows_v.at[i], g_hbm.at[pl.ds(chunk * C, C)], sem_s.at[i]
            )
        return carry

    lax.fori_loop(0, NGRP, body, 0)
    for i in range(K):
        pltpu.make_async_copy(
            rows_v.at[i], g_hbm.at[pl.ds(0, C)], sem_s.at[i]
        ).wait()

    @pl.when(wid < nchunks - NGRP * K * NW)
    def _():
        base = (NGRP * K * NW + wid) * C
        pltpu.sync_copy(col_hbm.at[pl.ds(base, C)], idx_v.at[0])
        pltpu.async_copy(xw_hbm.at[idx_v.at[0]], rows_v.at[0], sem_g.at[0]).wait()
        pltpu.sync_copy(rows_v.at[0], g_hbm.at[pl.ds(base, C)])


@functools.partial(
    pl.kernel,
    out_type=jax.ShapeDtypeStruct((NC, N_PAD, D), jnp.float32),
    mesh=_sc_mesh,
    scratch_types=[
        pltpu.VMEM((2 * KS, C), jnp.int32),
        pltpu.VMEM((KS, C, D), jnp.float32),
        pltpu.VMEM_SHARED((N_PAD, D), jnp.float32),
        pltpu.SemaphoreType.DMA((KS,)),
        pltpu.SemaphoreType.DMA((KS,)),
        pltpu.SemaphoreType.DMA((KS,)),
    ],
)
def _scatter_sc(
    f_hbm, row_hbm, col_hbm, zeros_hbm, out_hbm,
    idx_v, fbuf, flow_sh, sem_i, sem_f, sem_sc,
):
    cid = lax.axis_index("c")
    sid = lax.axis_index("s")

    pltpu.sync_copy(zeros_hbm, flow_sh.at[pl.ds(sid * ROWS_PER_TILE, ROWS_PER_TILE)])
    plsc.subcore_barrier()

    nchunks_half = (E // C) // NC

    def _select_trash(i):
        for ii in range(C // 16):
            r = idx_v[2 * i, pl.ds(ii * 16, 16)]
            cc = idx_v[2 * i + 1, pl.ds(ii * 16, 16)]
            trash = jnp.full((16,), N, jnp.int32)
            idx_v[2 * i, pl.ds(ii * 16, 16)] = jnp.where(r == cc, trash, r)

    def body(grp, carry):
        for i in range(KS):
            @pl.when(grp > 0)
            def _():
                pltpu.make_async_copy(
                    fbuf.at[i], flow_sh.at[idx_v.at[2 * i]], sem_sc.at[i]
                ).wait()
            chunk_local = sid + (grp * KS + i) * NS
            base = (cid * nchunks_half + chunk_local) * C
            pltpu.async_copy(row_hbm.at[pl.ds(base, C)], idx_v.at[2 * i], sem_i.at[i])
            pltpu.async_copy(col_hbm.at[pl.ds(base, C)], idx_v.at[2 * i + 1], sem_i.at[i])
            pltpu.async_copy(f_hbm.at[pl.ds(base, C)], fbuf.at[i], sem_f.at[i])
        for i in range(KS):
            pltpu.make_async_copy(
                row_hbm.at[pl.ds(0, C)], idx_v.at[2 * i], sem_i.at[i]
            ).wait()
            pltpu.make_async_copy(
                col_hbm.at[pl.ds(0, C)], idx_v.at[2 * i + 1], sem_i.at[i]
            ).wait()
            _select_trash(i)
            pltpu.make_async_copy(
                f_hbm.at[pl.ds(0, C)], fbuf.at[i], sem_f.at[i]
            ).wait()
            pltpu.async_copy(
                fbuf.at[i], flow_sh.at[idx_v.at[2 * i]], sem_sc.at[i], add=True
            )
        return carry

    lax.fori_loop(0, NGRPS, body, 0)
    for i in range(KS):
        pltpu.make_async_copy(
            fbuf.at[i], flow_sh.at[idx_v.at[2 * i]], sem_sc.at[i]
        ).wait()

    @pl.when(sid < nchunks_half - NGRPS * KS * NS)
    def _():
        base = (cid * nchunks_half + NGRPS * KS * NS + sid) * C
        pltpu.sync_copy(row_hbm.at[pl.ds(base, C)], idx_v.at[0])
        pltpu.sync_copy(col_hbm.at[pl.ds(base, C)], idx_v.at[1])
        _select_trash(0)
        pltpu.sync_copy(f_hbm.at[pl.ds(base, C)], fbuf.at[0])
        pltpu.sync_copy(fbuf.at[0], flow_sh.at[idx_v.at[0]], add=True)

    plsc.subcore_barrier()
    pltpu.sync_copy(
        flow_sh.at[pl.ds(sid * ROWS_PER_TILE, ROWS_PER_TILE)],
        out_hbm.at[cid, pl.ds(sid * ROWS_PER_TILE, ROWS_PER_TILE)],
    )


def kernel(x, edge_index, edge_attr, W1, b1, g1, bt1, Wn, bn, gn, btn):
    row = edge_index[0]
    col = edge_index[1]
    W1a = W1[:D]
    W1b = W1[D:]

    xw = pl.pallas_call(
        _xw_body,
        out_shape=jax.ShapeDtypeStruct((N, D), jnp.float32),
    )(x, W1a, b1.reshape(1, D))

    g = _gather_sc(xw, col)

    nblk = E // BE
    f = pl.pallas_call(
        _edge_body,
        grid=(nblk,),
        in_specs=[
            pl.BlockSpec((BE, D), lambda i: (i, 0)),
            pl.BlockSpec((BE, DE), lambda i: (i, 0)),
            pl.BlockSpec((DE, D), lambda i: (0, 0)),
            pl.BlockSpec((1, D), lambda i: (0, 0)),
            pl.BlockSpec((1, D), lambda i: (0, 0)),
        ],
        out_specs=pl.BlockSpec((BE, D), lambda i: (i, 0)),
        out_shape=jax.ShapeDtypeStruct((E, D), jnp.float32),
    )(g, edge_attr, W1b, g1.reshape(1, D), bt1.reshape(1, D))

    zeros = jnp.zeros((ROWS_PER_TILE, D), jnp.float32)
    partials = _scatter_sc(f, row, col, zeros)

    out = pl.pallas_call(
        _node_body,
        out_shape=jax.ShapeDtypeStruct((N, D), jnp.float32),
    )(partials, Wn, bn.reshape(1, D), gn.reshape(1, D), btn.reshape(1, D))
    return out


# speedup vs baseline: 1.4257x; 1.0457x over previous
import functools

import jax
import jax.numpy as jnp
from jax import lax
from jax.experimental import pallas as pl
from jax.experimental.pallas import tpu as pltpu
from jax.experimental.pallas import tpu_sc as plsc

N = 10000
E = 320000
D = 128
DE = 16

NC = 2
NS = 16
NW = NC * NS
C = 128

N_PAD = 10112
ROWS_PER_TILE = N_PAD // NS

BE = 8000

K = 6
NGRP = 13
KS = 3
NGRPS = 26


def _xw_body(x_ref, w_ref, b_ref, o_ref):
    o_ref[...] = (
        jnp.dot(x_ref[...], w_ref[...], preferred_element_type=jnp.float32)
        + b_ref[...]
    )


def _edge_body(g_ref, ea_ref, w_ref, g1_ref, bt1_ref, o_ref):
    f = g_ref[...] + jnp.dot(
        ea_ref[...], w_ref[...], preferred_element_type=jnp.float32
    )
    m = jnp.mean(f, axis=-1, keepdims=True)
    cgap = f - m
    v = jnp.mean(cgap * cgap, axis=-1, keepdims=True)
    h = cgap * lax.rsqrt(v + 1e-5) * g1_ref[...] + bt1_ref[...]
    o_ref[...] = jnp.maximum(h, 0.0)


def _node_body(p_ref, wn_ref, bn_ref, gn_ref, btn_ref, o_ref):
    ft = p_ref[0, :N, :] + p_ref[1, :N, :]
    f = jnp.dot(ft, wn_ref[...], preferred_element_type=jnp.float32) + bn_ref[...]
    m = jnp.mean(f, axis=-1, keepdims=True)
    cgap = f - m
    v = jnp.mean(cgap * cgap, axis=-1, keepdims=True)
    h = cgap * lax.rsqrt(v + 1e-5) * gn_ref[...] + btn_ref[...]
    o_ref[...] = jnp.maximum(h, 0.0)


_sc_mesh = plsc.VectorSubcoreMesh(core_axis_name="c", subcore_axis_name="s")


@functools.partial(
    pl.kernel,
    out_type=jax.ShapeDtypeStruct((E, D), jnp.float32),
    mesh=_sc_mesh,
    scratch_types=[
        pltpu.VMEM((K, C), jnp.int32),
        pltpu.VMEM((K, C, D), jnp.float32),
        pltpu.SemaphoreType.DMA((K,)),
        pltpu.SemaphoreType.DMA((K,)),
        pltpu.SemaphoreType.DMA((K,)),
    ],
)
def _gather_sc(xw_hbm, col_hbm, g_hbm, idx_v, rows_v, sem_i, sem_g, sem_s):
    wid = lax.axis_index("s") * NC + lax.axis_index("c")
    nchunks = E // C

    def body(grp, carry):
        for i in range(K):
            chunk = wid + (grp * K + i) * NW
            pltpu.async_copy(
                col_hbm.at[pl.ds(chunk * C, C)], idx_v.at[i], sem_i.at[i]
            )
        for i in range(K):
            @pl.when(grp > 0)
            def _():
                pltpu.make_async_copy(
                    rows_v.at[i], g_hbm.at[pl.ds(0, C)], sem_s.at[i]
                ).wait()
        for i in range(K):
            pltpu.make_async_copy(
                col_hbm.at[pl.ds(0, C)], idx_v.at[i], sem_i.at[i]
            ).wait()
            pltpu.async_copy(xw_hbm.at[idx_v.at[i]], rows_v.at[i], sem_g.at[i])
        for i in range(K):
            chunk = wid + (grp * K + i) * NW
            pltpu.make_async_copy(
                xw_hbm.at[idx_v.at[i]], rows_v.at[i], sem_g.at[i]
            ).wait()
            pltpu.async_copy(
                rows_v.at[i], g_hbm.at[pl.ds(chunk * C, C)], sem_s.at[i]
            )
        return carry

    lax.fori_loop(0, NGRP, body, 0)
    for i in range(K):
        pltpu.make_async_copy(
            rows_v.at[i], g_hbm.at[pl.ds(0, C)], sem_s.at[i]
        ).wait()

    @pl.when(wid < nchunks - NGRP * K * NW)
    def _():
        base = (NGRP * K * NW + wid) * C
        pltpu.sync_copy(col_hbm.at[pl.ds(base, C)], idx_v.at[0])
        pltpu.async_copy(xw_hbm.at[idx_v.at[0]], rows_v.at[0], sem_g.at[0]).wait()
        pltpu.sync_copy(rows_v.at[0], g_hbm.at[pl.ds(base, C)])


@functools.partial(
    pl.kernel,
    out_type=jax.ShapeDtypeStruct((NC, N_PAD, D), jnp.float32),
    mesh=_sc_mesh,
    scratch_types=[
        pltpu.VMEM((2 * KS, C), jnp.int32),
        pltpu.VMEM((KS, C, D), jnp.float32),
        pltpu.VMEM_SHARED((N_PAD, D), jnp.float32),
        pltpu.SemaphoreType.DMA((KS,)),
        pltpu.SemaphoreType.DMA((KS,)),
        pltpu.SemaphoreType.DMA((KS,)),
    ],
)
def _scatter_sc(
    f_hbm, row_hbm, col_hbm, zeros_hbm, out_hbm,
    idx_v, fbuf, flow_sh, sem_i, sem_f, sem_sc,
):
    cid = lax.axis_index("c")
    sid = lax.axis_index("s")

    pltpu.sync_copy(zeros_hbm, flow_sh.at[pl.ds(sid * ROWS_PER_TILE, ROWS_PER_TILE)])
    plsc.subcore_barrier()

    nchunks_half = (E // C) // NC

    def _select_trash(i):
        for ii in range(C // 16):
            r = idx_v[2 * i, pl.ds(ii * 16, 16)]
            cc = idx_v[2 * i + 1, pl.ds(ii * 16, 16)]
            trash = jnp.full((16,), N, jnp.int32)
            idx_v[2 * i, pl.ds(ii * 16, 16)] = jnp.where(r == cc, trash, r)

    def body(grp, carry):
        for i in range(KS):
            @pl.when(grp > 0)
            def _():
                pltpu.make_async_copy(
                    fbuf.at[i], flow_sh.at[idx_v.at[2 * i]], sem_sc.at[i]
                ).wait()
            chunk_local = sid + (grp * KS + i) * NS
            base = (cid * nchunks_half + chunk_local) * C
            pltpu.async_copy(row_hbm.at[pl.ds(base, C)], idx_v.at[2 * i], sem_i.at[i])
            pltpu.async_copy(col_hbm.at[pl.ds(base, C)], idx_v.at[2 * i + 1], sem_i.at[i])
            pltpu.async_copy(f_hbm.at[pl.ds(base, C)], fbuf.at[i], sem_f.at[i])
        for i in range(KS):
            pltpu.make_async_copy(
                row_hbm.at[pl.ds(0, C)], idx_v.at[2 * i], sem_i.at[i]
            ).wait()
            pltpu.make_async_copy(
                col_hbm.at[pl.ds(0, C)], idx_v.at[2 * i + 1], sem_i.at[i]
            ).wait()
            _select_trash(i)
            pltpu.make_async_copy(
                f_hbm.at[pl.ds(0, C)], fbuf.at[i], sem_f.at[i]
            ).wait()
            pltpu.async_copy(
                fbuf.at[i], flow_sh.at[idx_v.at[2 * i]], sem_sc.at[i], add=True
            )
        return carry

    lax.fori_loop(0, NGRPS, body, 0)
    for i in range(KS):
        pltpu.make_async_copy(
            fbuf.at[i], flow_sh.at[idx_v.at[2 * i]], sem_sc.at[i]
        ).wait()

    @pl.when(sid < nchunks_half - NGRPS * KS * NS)
    def _():
        base = (cid * nchunks_half + NGRPS * KS * NS + sid) * C
        pltpu.sync_copy(row_hbm.at[pl.ds(base, C)], idx_v.at[0])
        pltpu.sync_copy(col_hbm.at[pl.ds(base, C)], idx_v.at[1])
        _select_trash(0)
        pltpu.sync_copy(f_hbm.at[pl.ds(base, C)], fbuf.at[0])
        pltpu.sync_copy(fbuf.at[0], flow_sh.at[idx_v.at[0]], add=True)

    plsc.subcore_barrier()
    pltpu.sync_copy(
        flow_sh.at[pl.ds(sid * ROWS_PER_TILE, ROWS_PER_TILE)],
        out_hbm.at[cid, pl.ds(sid * ROWS_PER_TILE, ROWS_PER_TILE)],
    )


def kernel(x, edge_index, edge_attr, W1, b1, g1, bt1, Wn, bn, gn, btn):
    row = edge_index[0]
    col = edge_index[1]
    W1a = W1[:D]
    W1b = W1[D:]

    xw = pl.pallas_call(
        _xw_body,
        out_shape=jax.ShapeDtypeStruct((N, D), jnp.float32),
    )(x, W1a, b1.reshape(1, D))

    g = _gather_sc(xw, col)

    nblk = E // BE
    f = pl.pallas_call(
        _edge_body,
        grid=(nblk,),
        in_specs=[
            pl.BlockSpec((BE, D), lambda i: (i, 0)),
            pl.BlockSpec((BE, DE), lambda i: (i, 0)),
            pl.BlockSpec((DE, D), lambda i: (0, 0)),
            pl.BlockSpec((1, D), lambda i: (0, 0)),
            pl.BlockSpec((1, D), lambda i: (0, 0)),
        ],
        out_specs=pl.BlockSpec((BE, D), lambda i: (i, 0)),
        out_shape=jax.ShapeDtypeStruct((E, D), jnp.float32),
    )(g, edge_attr, W1b, g1.reshape(1, D), bt1.reshape(1, D))

    zeros = jnp.zeros((ROWS_PER_TILE, D), jnp.float32)
    partials = _scatter_sc(f, row, col, zeros)

    out = pl.pallas_call(
        _node_body,
        out_shape=jax.ShapeDtypeStruct((N, D), jnp.float32),
    )(partials, Wn, bn.reshape(1, D), gn.reshape(1, D), btn.reshape(1, D))
    return out
